# baseline (speedup 1.0000x reference)
import functools

import jax
import jax.numpy as jnp
from jax import lax
from jax.experimental import pallas as pl
from jax.experimental.pallas import tpu as pltpu
from jax.experimental.pallas import tpu_sc as plsc

N = 10000
F = 128
H = 128
V = 2
E = 320000
REG_COEF = 0.001

NC = 2
NS = 16
L = 16
CHUNK = 80
D = 3
TILE_EDGES = ((E // NS + CHUNK * D - 1) // (CHUNK * D)) * CHUNK * D
EPAD = TILE_EDGES * NS
NCHUNK = TILE_EDGES // CHUNK
ROWS_PER_TILE = 632


def _mm_body(x_ref, w_ref, out_ref):
    out_ref[0] = jnp.dot(x_ref[...], w_ref[0], preferred_element_type=jnp.float32)


def _xw(features, Wstack):
    return pl.pallas_call(
        _mm_body,
        grid=(V,),
        in_specs=[
            pl.BlockSpec((N, F), lambda i: (0, 0)),
            pl.BlockSpec((1, F, H), lambda i: (i, 0, 0)),
        ],
        out_specs=pl.BlockSpec((1, N, H), lambda i: (i, 0, 0)),
        out_shape=jax.ShapeDtypeStruct((V, N, H), jnp.float32),
    )(features, Wstack)


PROWS_PER_TILE = 640
NPB = PROWS_PER_TILE // CHUNK


def _sc_body(xw, src, dst, perm_hbm, zeros,
             out, xwp,
             sidx, didx, rows, pidx, acc, gsem, ssem, isem, dsem):
    c = lax.axis_index("c")
    s = lax.axis_index("s")

    prow0 = jnp.minimum(s * PROWS_PER_TILE, N - PROWS_PER_TILE)
    coff = c * N
    pltpu.sync_copy(perm_hbm.at[pl.ds(coff + prow0, PROWS_PER_TILE)], pidx)

    def _prow(b):
        return xwp.at[pl.ds(coff + prow0 + b * CHUNK, CHUNK)]

    def _pgather(b):
        return xw.at[pidx.at[pl.ds(b * CHUNK, CHUNK)]]

    pltpu.async_copy(_pgather(0), rows[0], gsem[0])
    pltpu.async_copy(_pgather(1), rows[1], gsem[1])
    for b in range(NPB):
        q = b % D
        pltpu.make_async_copy(_pgather(b), rows[q], gsem[q]).wait()
        pltpu.async_copy(rows[q], _prow(b), ssem[q])
        if b + 2 < NPB:
            qq = (b + 2) % D
            if b >= 1:
                pltpu.make_async_copy(rows[qq], _prow(b - 1), ssem[qq]).wait()
            pltpu.async_copy(_pgather(b + 2), rows[qq], gsem[qq])
    for b in range(NPB - 3, NPB):
        pltpu.make_async_copy(rows[b % D], _prow(b), ssem[b % D]).wait()
    plsc.subcore_barrier()

    row0 = jnp.minimum(s * ROWS_PER_TILE, N - ROWS_PER_TILE)
    ebase = c * EPAD + s * TILE_EDGES

    for phase, table in ((0, xw), (1, xwp)):
        pltpu.sync_copy(zeros.at[pl.ds(row0, ROWS_PER_TILE)],
                        acc.at[pl.ds(row0, ROWS_PER_TILE)])
        plsc.subcore_barrier()

        pltpu.sync_copy(src.at[pl.ds(ebase, CHUNK)], sidx[0])
        pltpu.async_copy(dst.at[pl.ds(ebase, CHUNK)], didx[0], dsem[0])
        pltpu.async_copy(table.at[sidx[0]], rows[0], gsem[0])
        pltpu.async_copy(src.at[pl.ds(ebase + CHUNK, CHUNK)], sidx[1],
                         isem[1])
        pltpu.async_copy(src.at[pl.ds(ebase + 2 * CHUNK, CHUNK)], sidx[2],
                         isem[2])
        pltpu.make_async_copy(src.at[pl.ds(ebase + CHUNK, CHUNK)], sidx[1],
                              isem[1]).wait()
        pltpu.async_copy(dst.at[pl.ds(ebase + CHUNK, CHUNK)], didx[1],
                         dsem[1])
        pltpu.async_copy(table.at[sidx[1]], rows[1], gsem[1])

        def block(t, carry):
            for k in range(D):
                q, q2 = k, (k + 2) % D
                m = t * D + k
                base = ebase + m * CHUNK

                @pl.when(m < NCHUNK - 2)
                def _():
                    pltpu.make_async_copy(
                        src.at[pl.ds(base + 2 * CHUNK, CHUNK)], sidx[q2],
                        isem[q2]).wait()

                @pl.when(m >= 1)
                def _():
                    pltpu.make_async_copy(rows[q2], acc.at[didx[q2]],
                                          ssem[q2]).wait()

                @pl.when(m < NCHUNK - 2)
                def _():
                    pltpu.async_copy(dst.at[pl.ds(base + 2 * CHUNK, CHUNK)],
                                     didx[q2], dsem[q2])
                    pltpu.async_copy(table.at[sidx[q2]], rows[q2], gsem[q2])

                pltpu.make_async_copy(table.at[sidx[q]], rows[q],
                                      gsem[q]).wait()
                pltpu.make_async_copy(dst.at[pl.ds(base, CHUNK)], didx[q],
                                      dsem[q]).wait()
                pltpu.async_copy(rows[q], acc.at[didx[q]], ssem[q], add=True)

                @pl.when(m < NCHUNK - 3)
                def _():
                    pltpu.async_copy(
                        src.at[pl.ds(base + 3 * CHUNK, CHUNK)], sidx[q],
                        isem[q])
            return carry

        lax.fori_loop(0, NCHUNK // D, block, 0)
        q = (NCHUNK - 1) % D
        pltpu.make_async_copy(rows[q], acc.at[didx[q]], ssem[q]).wait()
        plsc.subcore_barrier()
        outbase = (phase * V + c) * N + row0
        pltpu.sync_copy(acc.at[pl.ds(row0, ROWS_PER_TILE)],
                        out.at[pl.ds(outbase, ROWS_PER_TILE)])
        plsc.subcore_barrier()


def _segment_sums(xw_flat, srcv, dst_flat, permv, zeros):
    mesh = plsc.VectorSubcoreMesh(core_axis_name="c", subcore_axis_name="s")
    f = functools.partial(
        pl.kernel,
        mesh=mesh,
        out_type=(jax.ShapeDtypeStruct((2 * V * N, H), jnp.float32),
                  jax.ShapeDtypeStruct((V * N, H), jnp.float32)),
        scratch_types=[
            [pltpu.VMEM((CHUNK,), jnp.int32) for _ in range(D)],
            [pltpu.VMEM((CHUNK,), jnp.int32) for _ in range(D)],
            [pltpu.VMEM((CHUNK, H), jnp.float32) for _ in range(D)],
            pltpu.VMEM((PROWS_PER_TILE,), jnp.int32),
            pltpu.VMEM_SHARED((N + 8, H), jnp.float32),
            [pltpu.SemaphoreType.DMA for _ in range(D)],
            [pltpu.SemaphoreType.DMA for _ in range(D)],
            [pltpu.SemaphoreType.DMA for _ in range(D)],
            [pltpu.SemaphoreType.DMA for _ in range(D)],
        ],
    )(_sc_body)
    sums, _ = f(xw_flat, srcv, dst_flat, permv, zeros)
    return sums


def _loss_body(s_ref, wb_ref, bb_ref, hp_ref, out_ref):
    bb0 = bb_ref[0]
    xent = jnp.float32(0.0)
    hs = []
    for k in range(2 * V):
        hs.append(jnp.maximum(s_ref[k], 0.0))
    for v in range(V):
        h1, h2 = hs[v], hs[V + v]
        cvec = 1.0 / (1.0 + jnp.exp(-jnp.mean(h1, axis=0, keepdims=True)))
        w = jnp.sum(wb_ref[...] * cvec, axis=1, keepdims=True)
        s1 = jnp.dot(h1, w, preferred_element_type=jnp.float32) + bb0
        s2 = jnp.dot(h2, w, preferred_element_type=jnp.float32) + bb0
        t1 = jnp.maximum(s1, 0.0) - s1 + jnp.log1p(jnp.exp(-jnp.abs(s1)))
        t2 = jnp.maximum(s2, 0.0) + jnp.log1p(jnp.exp(-jnp.abs(s2)))
        xent = xent + (jnp.sum(t1) + jnp.sum(t2)) / jnp.float32(2 * N)
    h1a = 0.5 * (hs[0] + hs[1])
    h2a = 0.5 * (hs[2] + hs[3])
    hp = hp_ref[...]
    pos = jnp.sum((hp - h1a) ** 2)
    neg = jnp.sum((hp - h2a) ** 2)
    total = xent + jnp.float32(REG_COEF) * (pos - neg)
    out_ref[...] = jnp.reshape(total, (1, 1))


def _loss(sums, Wb, bb, Hparam):
    return pl.pallas_call(
        _loss_body,
        out_shape=jax.ShapeDtypeStruct((1, 1), jnp.float32),
    )(sums, Wb, bb, Hparam)


def kernel(features, W0, W1, Wb, bb, Hparam, edge_index_0, edge_index_1, perm):
    xw = _xw(features, jnp.stack([W0, W1]))
    xw_flat = xw.reshape(V * N, H)

    def _pad_tiles(x, fill):
        per_tile = x.reshape(NS, E // NS)
        pad = jnp.full((NS, TILE_EDGES - E // NS), fill, jnp.int32)
        return jnp.concatenate([per_tile, pad], axis=1).reshape(-1)

    src_v = jnp.concatenate([_pad_tiles(edge_index_0[0], 0),
                             _pad_tiles(edge_index_1[0] + N, 0)])
    dst_flat = jnp.concatenate([_pad_tiles(edge_index_0[1], N),
                                _pad_tiles(edge_index_1[1], N)])
    perm_v = jnp.concatenate([perm, perm + N])
    zeros = jnp.zeros((N, H), jnp.float32)
    sums = _segment_sums(xw_flat, src_v, dst_flat, perm_v, zeros)
    loss = _loss(sums.reshape(2 * V, N, H), Wb, bb, Hparam)
    return loss.reshape(())


# speedup vs baseline: 1.5432x; 1.5432x over previous
import functools

import jax
import jax.numpy as jnp
from jax import lax
from jax.experimental import pallas as pl
from jax.experimental.pallas import tpu as pltpu
from jax.experimental.pallas import tpu_sc as plsc

N = 10000
F = 128
H = 128
V = 2
E = 320000
REG_COEF = 0.001

NC = 2
NS = 16
L = 16
CHUNK = 80
D = 2
TILE_EDGES = ((E // NS + CHUNK - 1) // CHUNK) * CHUNK
EPAD = TILE_EDGES * NS
NCHUNK = TILE_EDGES // CHUNK
ROWS_PER_TILE = 632


def _mm_body(x_ref, w_ref, out_ref):
    out_ref[0] = jnp.dot(x_ref[...], w_ref[0], preferred_element_type=jnp.float32)


def _xw(features, Wstack):
    return pl.pallas_call(
        _mm_body,
        grid=(V,),
        in_specs=[
            pl.BlockSpec((N, F), lambda i: (0, 0)),
            pl.BlockSpec((1, F, H), lambda i: (i, 0, 0)),
        ],
        out_specs=pl.BlockSpec((1, N, H), lambda i: (i, 0, 0)),
        out_shape=jax.ShapeDtypeStruct((V, N, H), jnp.float32),
    )(features, Wstack)


PROWS_PER_TILE = 640
NPB = PROWS_PER_TILE // CHUNK


def _sc_body(xw, src, dst, perm_hbm, zeros,
             out, xwp,
             sidx, didx, rows, acc, gsem, ssem, isem, dsem):
    c = lax.axis_index("c")
    s = lax.axis_index("s")

    prow0 = jnp.minimum(s * PROWS_PER_TILE, N - PROWS_PER_TILE)
    coff = c * N
    for b in range(NPB):
        p = b % D
        pltpu.sync_copy(perm_hbm.at[pl.ds(coff + prow0 + b * CHUNK, CHUNK)],
                        sidx[p])
        pltpu.async_copy(xw.at[sidx[p]], rows[p], gsem[p]).wait()
        pltpu.async_copy(rows[p],
                         xwp.at[pl.ds(coff + prow0 + b * CHUNK, CHUNK)],
                         ssem[p]).wait()
    plsc.subcore_barrier()

    row0 = jnp.minimum(s * ROWS_PER_TILE, N - ROWS_PER_TILE)
    ebase = c * EPAD + s * TILE_EDGES

    for phase, table in ((0, xw), (1, xwp)):
        pltpu.sync_copy(zeros.at[pl.ds(row0, ROWS_PER_TILE)],
                        acc.at[pl.ds(row0, ROWS_PER_TILE)])
        plsc.subcore_barrier()

        pltpu.sync_copy(src.at[pl.ds(ebase, CHUNK)], sidx[0])
        pltpu.async_copy(dst.at[pl.ds(ebase, CHUNK)], didx[0], dsem[0])
        pltpu.async_copy(table.at[sidx[0]], rows[0], gsem[0])
        pltpu.async_copy(src.at[pl.ds(ebase + CHUNK, CHUNK)], sidx[1],
                         isem[1])

        def block(t, carry):
            for k in range(D):
                q, qn = k, (k + 1) % D
                m = t * D + k
                base = ebase + m * CHUNK

                @pl.when(m < NCHUNK - 1)
                def _():
                    pltpu.make_async_copy(
                        src.at[pl.ds(base + CHUNK, CHUNK)], sidx[qn],
                        isem[qn]).wait()

                @pl.when(m >= 1)
                def _():
                    pltpu.make_async_copy(rows[qn], acc.at[didx[qn]],
                                          ssem[qn]).wait()

                @pl.when(m < NCHUNK - 1)
                def _():
                    pltpu.async_copy(dst.at[pl.ds(base + CHUNK, CHUNK)],
                                     didx[qn], dsem[qn])
                    pltpu.async_copy(table.at[sidx[qn]], rows[qn], gsem[qn])

                pltpu.make_async_copy(table.at[sidx[q]], rows[q],
                                      gsem[q]).wait()
                pltpu.make_async_copy(dst.at[pl.ds(base, CHUNK)], didx[q],
                                      dsem[q]).wait()
                pltpu.async_copy(rows[q], acc.at[didx[q]], ssem[q], add=True)

                @pl.when(m < NCHUNK - 2)
                def _():
                    pltpu.async_copy(
                        src.at[pl.ds(base + 2 * CHUNK, CHUNK)], sidx[q],
                        isem[q])
            return carry

        lax.fori_loop(0, NCHUNK // D, block, 0)
        for m in range(NCHUNK - D + 1, NCHUNK):
            q = m % D
            pltpu.make_async_copy(rows[q], acc.at[didx[q]], ssem[q]).wait()
        plsc.subcore_barrier()
        outbase = (phase * V + c) * N + row0
        pltpu.sync_copy(acc.at[pl.ds(row0, ROWS_PER_TILE)],
                        out.at[pl.ds(outbase, ROWS_PER_TILE)])
        plsc.subcore_barrier()


def _segment_sums(xw_flat, srcv, dst_flat, permv, zeros):
    mesh = plsc.VectorSubcoreMesh(core_axis_name="c", subcore_axis_name="s")
    f = functools.partial(
        pl.kernel,
        mesh=mesh,
        out_type=(jax.ShapeDtypeStruct((2 * V * N, H), jnp.float32),
                  jax.ShapeDtypeStruct((V * N, H), jnp.float32)),
        scratch_types=[
            [pltpu.VMEM((CHUNK,), jnp.int32) for _ in range(D)],
            [pltpu.VMEM((CHUNK,), jnp.int32) for _ in range(D)],
            [pltpu.VMEM((CHUNK, H), jnp.float32) for _ in range(D)],
            pltpu.VMEM_SHARED((N + 8, H), jnp.float32),
            [pltpu.SemaphoreType.DMA for _ in range(D)],
            [pltpu.SemaphoreType.DMA for _ in range(D)],
            [pltpu.SemaphoreType.DMA for _ in range(D)],
            [pltpu.SemaphoreType.DMA for _ in range(D)],
        ],
    )(_sc_body)
    sums, _ = f(xw_flat, srcv, dst_flat, permv, zeros)
    return sums


def _loss_body(s_ref, wb_ref, bb_ref, hp_ref, out_ref):
    bb0 = bb_ref[0]
    xent = jnp.float32(0.0)
    hs = []
    for k in range(2 * V):
        hs.append(jnp.maximum(s_ref[k], 0.0))
    for v in range(V):
        h1, h2 = hs[v], hs[V + v]
        cvec = 1.0 / (1.0 + jnp.exp(-jnp.mean(h1, axis=0, keepdims=True)))
        w = jnp.sum(wb_ref[...] * cvec, axis=1, keepdims=True)
        s1 = jnp.dot(h1, w, preferred_element_type=jnp.float32) + bb0
        s2 = jnp.dot(h2, w, preferred_element_type=jnp.float32) + bb0
        t1 = jnp.maximum(s1, 0.0) - s1 + jnp.log1p(jnp.exp(-jnp.abs(s1)))
        t2 = jnp.maximum(s2, 0.0) + jnp.log1p(jnp.exp(-jnp.abs(s2)))
        xent = xent + (jnp.sum(t1) + jnp.sum(t2)) / jnp.float32(2 * N)
    h1a = 0.5 * (hs[0] + hs[1])
    h2a = 0.5 * (hs[2] + hs[3])
    hp = hp_ref[...]
    pos = jnp.sum((hp - h1a) ** 2)
    neg = jnp.sum((hp - h2a) ** 2)
    total = xent + jnp.float32(REG_COEF) * (pos - neg)
    out_ref[...] = jnp.reshape(total, (1, 1))


def _loss(sums, Wb, bb, Hparam):
    return pl.pallas_call(
        _loss_body,
        out_shape=jax.ShapeDtypeStruct((1, 1), jnp.float32),
    )(sums, Wb, bb, Hparam)


def kernel(features, W0, W1, Wb, bb, Hparam, edge_index_0, edge_index_1, perm):
    xw = _xw(features, jnp.stack([W0, W1]))
    xw_flat = xw.reshape(V * N, H)

    def _pad_tiles(x, fill):
        per_tile = x.reshape(NS, E // NS)
        pad = jnp.full((NS, TILE_EDGES - E // NS), fill, jnp.int32)
        return jnp.concatenate([per_tile, pad], axis=1).reshape(-1)

    src_v = jnp.concatenate([_pad_tiles(edge_index_0[0], 0),
                             _pad_tiles(edge_index_1[0] + N, 0)])
    dst_flat = jnp.concatenate([_pad_tiles(edge_index_0[1], N),
                                _pad_tiles(edge_index_1[1], N)])
    perm_v = jnp.concatenate([perm, perm + N])
    zeros = jnp.zeros((N, H), jnp.float32)
    sums = _segment_sums(xw_flat, src_v, dst_flat, perm_v, zeros)
    loss = _loss(sums.reshape(2 * V, N, H), Wb, bb, Hparam)
    return loss.reshape(())
